# Initial kernel scaffold; baseline (speedup 1.0000x reference)
import functools

import jax
import jax.numpy as jnp
from jax import lax
from jax.experimental import pallas as pl
from jax.experimental.pallas import tpu as pltpu
from jax.experimental.pallas import tpu_sc as plsc

_CHUNK = 128
_NTILES = 32
_SUBCORES = 16


def _sc_edge_aggregate(table, src, dst, npad):
    v, width = table.shape
    e = src.shape[0]
    nchunks = e // _CHUNK
    rows_per_tile = npad // _SUBCORES
    nfull = nchunks // _NTILES
    extra = nchunks % _NTILES
    niter = nfull + (1 if extra else 0)
    zeros = jnp.zeros((npad, width), jnp.float32)

    mesh = plsc.VectorSubcoreMesh(core_axis_name="c", subcore_axis_name="s")

    @functools.partial(
        pl.kernel,
        out_type=jax.ShapeDtypeStruct((2 * npad, width), jnp.float32),
        mesh=mesh,
        scratch_types=[
            pltpu.VMEM((_CHUNK,), jnp.int32),
            pltpu.VMEM((_CHUNK,), jnp.int32),
            pltpu.VMEM((_CHUNK, width), jnp.float32),
            pltpu.VMEM_SHARED((npad, width), jnp.float32),
            pltpu.SemaphoreType.DMA,
        ],
    )
    def agg_kernel(table_hbm, src_hbm, dst_hbm, z_hbm, out_hbm,
                   sidx, didx, rows, shared, sem):
        cid = lax.axis_index("c")
        sid = lax.axis_index("s")
        wid = sid * 2 + cid

        base_r = sid * rows_per_tile
        pltpu.sync_copy(z_hbm.at[pl.ds(base_r, rows_per_tile)],
                        shared.at[pl.ds(base_r, rows_per_tile)])
        plsc.subcore_barrier()

        @pl.loop(0, niter)
        def _(j):
            c = j * _NTILES + wid

            @pl.when(c < nchunks)
            def _():
                base = c * _CHUNK
                pltpu.sync_copy(src_hbm.at[pl.ds(base, _CHUNK)], sidx)
                pltpu.sync_copy(dst_hbm.at[pl.ds(base, _CHUNK)], didx)
                pltpu.async_copy(table_hbm.at[sidx], rows, sem).wait()
                pltpu.sync_copy(rows, shared.at[didx], add=True)

        plsc.subcore_barrier()
        pltpu.sync_copy(shared.at[pl.ds(base_r, rows_per_tile)],
                        out_hbm.at[pl.ds(cid * npad + base_r, rows_per_tile)])

    return agg_kernel(table, src, dst, zeros)


def _tc_layer1_body(pa_ref, w_ref, b_ref, h_ref):
    npad = pa_ref.shape[0] // 2
    s = pa_ref[:npad, :] + pa_ref[npad:, :]
    agg = s[:, :128]
    deg = jnp.maximum(s[:, 128:129], 1.0)
    z = jnp.dot(agg / deg, w_ref[...], precision=lax.Precision.HIGHEST,
                preferred_element_type=jnp.float32)
    h_ref[...] = jnp.maximum(z + b_ref[...], 0.0)


def _tc_layer2_body(pb_ref, dc_ref, batch_ref, w_ref, b_ref, wo_ref, bo_ref,
                    out_ref, *, num_graphs):
    npad = pb_ref.shape[0] // 2
    s = pb_ref[:npad, :] + pb_ref[npad:, :]
    deg = jnp.maximum(dc_ref[:npad, 0:1] + dc_ref[npad:, 0:1], 1.0)
    h = jnp.maximum(
        jnp.dot(s / deg, w_ref[...], precision=lax.Precision.HIGHEST,
                preferred_element_type=jnp.float32) + b_ref[...], 0.0)
    b = batch_ref[...]
    gids = lax.broadcasted_iota(jnp.int32, (1, num_graphs), 1)
    pt = (b == gids).astype(jnp.float32)
    counts = jnp.maximum(jnp.sum(pt, axis=0), 1.0)
    hg = lax.dot_general(pt, h, (((0,), (0,)), ((), ())),
                         precision=lax.Precision.HIGHEST,
                         preferred_element_type=jnp.float32)
    hg = hg / counts[:, None]
    out_ref[...] = jnp.dot(hg, wo_ref[...], precision=lax.Precision.HIGHEST,
                           preferred_element_type=jnp.float32) + bo_ref[...]


def kernel(x, edge_index, batch, W1, b1, W2, b2, Wout, bout):
    n, d = x.shape
    num_graphs = 64
    npad = ((n + _NTILES * 8 - 1) // (_NTILES * 8)) * (_NTILES * 8)

    src = edge_index[0]
    dst = edge_index[1]

    xa = jnp.concatenate([x, jnp.ones((n, 16), jnp.float32)], axis=1)
    pa = _sc_edge_aggregate(xa, src, dst, npad)
    h1 = pl.pallas_call(
        _tc_layer1_body,
        out_shape=jax.ShapeDtypeStruct((npad, 128), jnp.float32),
    )(pa, W1, b1)

    pb = _sc_edge_aggregate(h1, src, dst, npad)

    dcols = pa[:, 128:144]
    batch_p = jnp.concatenate(
        [batch, jnp.full((npad - n,), num_graphs, jnp.int32)]).reshape(npad, 1)
    out = pl.pallas_call(
        functools.partial(_tc_layer2_body, num_graphs=num_graphs),
        out_shape=jax.ShapeDtypeStruct((num_graphs, 128), jnp.float32),
    )(pb, dcols, batch_p, W2, b2, Wout, bout)
    return out


# speedup vs baseline: 6.6653x; 6.6653x over previous
import dataclasses
import functools

import jax
import jax.numpy as jnp
from jax import lax
from jax.experimental import pallas as pl
from jax.experimental.pallas import tpu as pltpu
from jax.experimental.pallas import tpu_sc as plsc

_CHUNK = 128
_NTILES = 32
_SUBCORES = 16
_LANES = 16


def _sc_edge_aggregate(table, src, dst, npad, with_deg):
    v, width = table.shape
    e = src.shape[0]
    nchunks = e // _CHUNK
    rows_per_tile = npad // _SUBCORES
    nfull = nchunks // _NTILES
    extra = nchunks % _NTILES
    niter = nfull + (1 if extra else 0)
    zeros = jnp.zeros((npad, width), jnp.float32)

    mesh = plsc.VectorSubcoreMesh(core_axis_name="c", subcore_axis_name="s")

    out_type = [jax.ShapeDtypeStruct((2 * npad, width), jnp.float32)]
    scratch = [
        pltpu.VMEM((_CHUNK,), jnp.int32),
        pltpu.VMEM((_CHUNK,), jnp.int32),
        pltpu.VMEM((_CHUNK, width), jnp.float32),
        pltpu.VMEM_SHARED((npad, width), jnp.float32),
        pltpu.SemaphoreType.DMA,
    ]
    if with_deg:
        out_type.append(jax.ShapeDtypeStruct((_NTILES, npad), jnp.float32))
        scratch.append(pltpu.VMEM((npad,), jnp.float32))

    cp = pltpu.CompilerParams()
    if "needs_layout_passes" in pltpu.CompilerParams.__dataclass_fields__:
        cp = dataclasses.replace(cp, needs_layout_passes=False)

    @functools.partial(pl.kernel, out_type=out_type, mesh=mesh,
                       scratch_types=scratch, compiler_params=cp)
    def agg_kernel(table_hbm, src_hbm, dst_hbm, z_hbm, *refs):
        if with_deg:
            out_hbm, deg_hbm, sidx, didx, rows, shared, sem, ldeg = refs
        else:
            out_hbm, sidx, didx, rows, shared, sem = refs
        cid = lax.axis_index("c")
        sid = lax.axis_index("s")
        wid = sid * 2 + cid

        base_r = sid * rows_per_tile
        pltpu.sync_copy(z_hbm.at[pl.ds(base_r, rows_per_tile)],
                        shared.at[pl.ds(base_r, rows_per_tile)])
        if with_deg:
            zv = jnp.zeros((_LANES,), jnp.float32)

            @pl.loop(0, npad // _LANES)
            def _(i):
                ldeg[pl.ds(i * _LANES, _LANES)] = zv

        plsc.subcore_barrier()

        @pl.loop(0, niter)
        def _(j):
            c = j * _NTILES + wid

            @pl.when(c < nchunks)
            def _():
                base = c * _CHUNK
                pltpu.sync_copy(src_hbm.at[pl.ds(base, _CHUNK)], sidx)
                pltpu.sync_copy(dst_hbm.at[pl.ds(base, _CHUNK)], didx)
                pltpu.async_copy(table_hbm.at[sidx], rows, sem).wait()
                pltpu.sync_copy(rows, shared.at[didx], add=True)
                if with_deg:
                    ones = jnp.ones((_LANES,), jnp.float32)
                    for k in range(_CHUNK // _LANES):
                        idxv = didx[pl.ds(k * _LANES, _LANES)]
                        plsc.addupdate_scatter(ldeg, [idxv], ones)

        plsc.subcore_barrier()
        pltpu.sync_copy(shared.at[pl.ds(base_r, rows_per_tile)],
                        out_hbm.at[pl.ds(cid * npad + base_r, rows_per_tile)])
        if with_deg:
            pltpu.sync_copy(ldeg, deg_hbm.at[wid])

    return agg_kernel(table, src, dst, zeros)


def _sum_deg(dp, npad):
    ones = jnp.ones((_NTILES, 1), jnp.float32)
    deg = lax.dot_general(dp, ones, (((0,), (0,)), ((), ())),
                          precision=lax.Precision.HIGHEST,
                          preferred_element_type=jnp.float32)
    return jnp.maximum(deg, 1.0)


def _tc_layer1_body(pa_ref, dp_ref, w_ref, b_ref, h_ref):
    npad = pa_ref.shape[0] // 2
    s = pa_ref[:npad, :] + pa_ref[npad:, :]
    deg = _sum_deg(dp_ref[...], npad)
    z = jnp.dot(s / deg, w_ref[...], precision=lax.Precision.HIGHEST,
                preferred_element_type=jnp.float32)
    h_ref[...] = jnp.maximum(z + b_ref[...], 0.0)


def _tc_layer2_body(pb_ref, dp_ref, batch_ref, w_ref, b_ref, wo_ref, bo_ref,
                    out_ref, *, num_graphs):
    npad = pb_ref.shape[0] // 2
    s = pb_ref[:npad, :] + pb_ref[npad:, :]
    deg = _sum_deg(dp_ref[...], npad)
    h = jnp.maximum(
        jnp.dot(s / deg, w_ref[...], precision=lax.Precision.HIGHEST,
                preferred_element_type=jnp.float32) + b_ref[...], 0.0)
    b = batch_ref[...]
    gids = lax.broadcasted_iota(jnp.int32, (1, num_graphs), 1)
    pt = (b == gids).astype(jnp.float32)
    counts = jnp.maximum(jnp.sum(pt, axis=0), 1.0)
    hg = lax.dot_general(pt, h, (((0,), (0,)), ((), ())),
                         precision=lax.Precision.HIGHEST,
                         preferred_element_type=jnp.float32)
    hg = hg / counts[:, None]
    out_ref[...] = jnp.dot(hg, wo_ref[...], precision=lax.Precision.HIGHEST,
                           preferred_element_type=jnp.float32) + bo_ref[...]


def kernel(x, edge_index, batch, W1, b1, W2, b2, Wout, bout):
    n, d = x.shape
    num_graphs = 64
    npad = ((n + _NTILES * 8 - 1) // (_NTILES * 8)) * (_NTILES * 8)

    src = edge_index[0]
    dst = edge_index[1]

    pa, dp = _sc_edge_aggregate(x, src, dst, npad, with_deg=True)
    h1 = pl.pallas_call(
        _tc_layer1_body,
        out_shape=jax.ShapeDtypeStruct((npad, 128), jnp.float32),
    )(pa, dp, W1, b1)

    (pb,) = _sc_edge_aggregate(h1, src, dst, npad, with_deg=False)

    batch_p = jnp.concatenate(
        [batch, jnp.full((npad - n,), num_graphs, jnp.int32)]).reshape(npad, 1)
    out = pl.pallas_call(
        functools.partial(_tc_layer2_body, num_graphs=num_graphs),
        out_shape=jax.ShapeDtypeStruct((num_graphs, 128), jnp.float32),
    )(pb, dp, batch_p, W2, b2, Wout, bout)
    return out
